# Initial kernel scaffold; baseline (speedup 1.0000x reference)
import jax
import jax.numpy as jnp
from jax.experimental import pallas as pl

N_NODES = 10000
N_EDGES = 160000
D_FEAT = 128
N_BASIS = 4
CUTOFF = 5.0
GAMMA = 4.0

_BLK = 2000


def _edge_kernel(disp_ref, w_ref, nfe_ref, out_ref):
    disp = disp_ref[...]
    w = w_ref[...]
    nfe = nfe_ref[...]
    d2 = jnp.sum(disp * disp, axis=-1, keepdims=True)
    mask = w > 0.5
    safe_d2 = jnp.where(mask, d2, 1.0)
    inv_d = jax.lax.rsqrt(safe_d2)
    d = safe_d2 * inv_d
    mu = jnp.linspace(0.0, CUTOFF, N_BASIS, dtype=jnp.float32)
    rbf = jnp.exp(-GAMMA * (d - mu[None, :]) ** 2)
    unit = disp * inv_d
    coef = unit[:, :, None] * rbf[:, None, :] * w[:, :, None]
    out = coef[:, :, :, None] * nfe[:, None, None, :]
    out_ref[...] = out.reshape(out.shape[0], 3, N_BASIS * D_FEAT)


def kernel(pos, node_features, edge_idx):
    src = edge_idx[0].astype(jnp.int32)
    dst = edge_idx[1].astype(jnp.int32)
    disp = pos[dst] - pos[src]
    w = (src != dst).astype(jnp.float32)[:, None]
    nfe = node_features[dst]

    edge_vals = pl.pallas_call(
        _edge_kernel,
        grid=(N_EDGES // _BLK,),
        in_specs=[
            pl.BlockSpec((_BLK, 3), lambda i: (i, 0)),
            pl.BlockSpec((_BLK, 1), lambda i: (i, 0)),
            pl.BlockSpec((_BLK, D_FEAT), lambda i: (i, 0)),
        ],
        out_specs=pl.BlockSpec((_BLK, 3, N_BASIS * D_FEAT), lambda i: (i, 0, 0)),
        out_shape=jax.ShapeDtypeStruct((N_EDGES, 3, N_BASIS * D_FEAT), jnp.float32),
    )(disp, w, nfe)

    node_vecs = jax.ops.segment_sum(edge_vals, src, num_segments=N_NODES)
    x = node_vecs[:, 0, :]
    y = node_vecs[:, 1, :]
    z = node_vecs[:, 2, :]
    inv_sqrt2 = 1.0 / jnp.sqrt(jnp.asarray(2.0, dtype=jnp.float32))
    m_minus = jnp.stack([x * inv_sqrt2, -y * inv_sqrt2], axis=-1)
    m_zero = jnp.stack([z, jnp.zeros_like(z)], axis=-1)
    m_plus = jnp.stack([-x * inv_sqrt2, -y * inv_sqrt2], axis=-1)
    return jnp.stack([m_minus, m_zero, m_plus], axis=-3)


# speedup vs baseline: 1.0435x; 1.0435x over previous
import jax
import jax.numpy as jnp
from jax.experimental import pallas as pl

N_NODES = 10000
N_EDGES = 160000
D_FEAT = 128
N_BASIS = 4
CUTOFF = 5.0
GAMMA = 4.0

_BLK = 2000


def _edge_kernel(disp_ref, w_ref, nfe_ref, out_ref):
    disp = disp_ref[...]
    w = w_ref[...]
    nfe = nfe_ref[...]
    d2 = jnp.sum(disp * disp, axis=-1, keepdims=True)
    mask = w > 0.5
    safe_d2 = jnp.where(mask, d2, 1.0)
    inv_d = jax.lax.rsqrt(safe_d2)
    d = safe_d2 * inv_d
    mu = jax.lax.broadcasted_iota(jnp.int32, (1, N_BASIS), 1).astype(
        jnp.float32) * (CUTOFF / (N_BASIS - 1))
    rbf = jnp.exp(-GAMMA * (d - mu) ** 2)
    unit = disp * inv_d
    coef = unit[:, :, None] * rbf[:, None, :] * w[:, :, None]
    out = coef[:, :, :, None] * nfe[:, None, None, :]
    out_ref[...] = out.reshape(out.shape[0], 3, N_BASIS * D_FEAT)


def kernel(pos, node_features, edge_idx):
    src = edge_idx[0].astype(jnp.int32)
    dst = edge_idx[1].astype(jnp.int32)
    disp = pos[dst] - pos[src]
    w = (src != dst).astype(jnp.float32)[:, None]
    nfe = node_features[dst]

    edge_vals = pl.pallas_call(
        _edge_kernel,
        grid=(N_EDGES // _BLK,),
        in_specs=[
            pl.BlockSpec((_BLK, 3), lambda i: (i, 0)),
            pl.BlockSpec((_BLK, 1), lambda i: (i, 0)),
            pl.BlockSpec((_BLK, D_FEAT), lambda i: (i, 0)),
        ],
        out_specs=pl.BlockSpec((_BLK, 3, N_BASIS * D_FEAT), lambda i: (i, 0, 0)),
        out_shape=jax.ShapeDtypeStruct((N_EDGES, 3, N_BASIS * D_FEAT), jnp.float32),
    )(disp, w, nfe)

    node_vecs = jax.ops.segment_sum(edge_vals, src, num_segments=N_NODES)
    x = node_vecs[:, 0, :]
    y = node_vecs[:, 1, :]
    z = node_vecs[:, 2, :]
    inv_sqrt2 = 1.0 / jnp.sqrt(jnp.asarray(2.0, dtype=jnp.float32))
    m_minus = jnp.stack([x * inv_sqrt2, -y * inv_sqrt2], axis=-1)
    m_zero = jnp.stack([z, jnp.zeros_like(z)], axis=-1)
    m_plus = jnp.stack([-x * inv_sqrt2, -y * inv_sqrt2], axis=-1)
    return jnp.stack([m_minus, m_zero, m_plus], axis=-3)


# speedup vs baseline: 6.4921x; 6.2212x over previous
import functools

import jax
import jax.numpy as jnp
from jax import lax
from jax.experimental import pallas as pl
from jax.experimental.pallas import tpu as pltpu
from jax.experimental.pallas import tpu_sc as plsc

N_NODES = 10000
N_EDGES = 160000
D_FEAT = 128
N_BASIS = 4
CUTOFF = 5.0
GAMMA = 4.0

NC = 2
NS = 16
NW = NC * NS
SC_CHUNK = 128
E_PER_W = 5120
EP = NW * E_PER_W
N_PAD = 10240

B = 256
NB = N_EDGES // B
NV = 3 * N_BASIS * D_FEAT

_MU = [0.0, CUTOFF / 3.0, 2.0 * CUTOFF / 3.0, CUTOFF]


def _sc_edge_kernel(posx_h, posy_h, posz_h, srcs_h, dsts_h, nf_h,
                    coef_h, nfe_h,
                    posx_v, posy_v, posz_v, src_v, dst_v,
                    coef_v, rows_v, sem):
    wid = lax.axis_index("s") * NC + lax.axis_index("c")
    pltpu.sync_copy(posx_h, posx_v)
    pltpu.sync_copy(posy_h, posy_v)
    pltpu.sync_copy(posz_h, posz_v)

    def chunk_body(ci, _):
        base = wid * E_PER_W + ci * SC_CHUNK
        pltpu.sync_copy(srcs_h.at[pl.ds(base, SC_CHUNK)], src_v)
        pltpu.sync_copy(dsts_h.at[pl.ds(base, SC_CHUNK)], dst_v)
        pltpu.async_copy(nf_h.at[dst_v], rows_v, sem).wait()

        for g in range(SC_CHUNK // 16):
            s16 = src_v[pl.ds(g * 16, 16)]
            d16 = dst_v[pl.ds(g * 16, 16)]
            xs = plsc.load_gather(posx_v, [s16])
            ys = plsc.load_gather(posy_v, [s16])
            zs = plsc.load_gather(posz_v, [s16])
            xd = plsc.load_gather(posx_v, [d16])
            yd = plsc.load_gather(posy_v, [d16])
            zd = plsc.load_gather(posz_v, [d16])
            dx = xd - xs
            dy = yd - ys
            dz = zd - zs
            w = jnp.where(s16 != d16, 1.0, 0.0).astype(jnp.float32)
            d2 = jnp.where(s16 != d16, dx * dx + dy * dy + dz * dz, 1.0)
            y = plsc.bitcast(
                jnp.int32(0x5F3759DF) - (plsc.bitcast(d2, jnp.int32) >> 1),
                jnp.float32)
            for _ in range(3):
                y = y * (1.5 - 0.5 * d2 * y * y)
            d = d2 * y
            ux = dx * y * w
            uy = dy * y * w
            uz = dz * y * w
            rows = lax.iota(jnp.int32, 16) + g * 16
            for k in range(N_BASIS):
                dk = d - _MU[k]
                rbf = jnp.exp(-GAMMA * dk * dk)
                for i, u in enumerate((ux, uy, uz)):
                    col = jnp.full((16,), i * N_BASIS + k, jnp.int32)
                    plsc.store_scatter(coef_v, [rows, col], u * rbf)

        pltpu.sync_copy(coef_v, coef_h.at[pl.ds(base, SC_CHUNK)])
        pltpu.sync_copy(rows_v, nfe_h.at[pl.ds(base, SC_CHUNK)])

    lax.fori_loop(0, E_PER_W // SC_CHUNK, chunk_body, None)


def _sc_edge_phase(posx, posy, posz, srcs, dsts, nf):
    mesh = plsc.VectorSubcoreMesh(core_axis_name="c", subcore_axis_name="s")
    f = pl.kernel(
        _sc_edge_kernel,
        mesh=mesh,
        compiler_params=pltpu.CompilerParams(
            use_tc_tiling_on_sc=False, needs_layout_passes=False),
        out_type=(
            jax.ShapeDtypeStruct((EP, 16), jnp.float32),
            jax.ShapeDtypeStruct((EP, D_FEAT), jnp.float32),
        ),
        scratch_types=[
            pltpu.VMEM((N_PAD,), jnp.float32),
            pltpu.VMEM((N_PAD,), jnp.float32),
            pltpu.VMEM((N_PAD,), jnp.float32),
            pltpu.VMEM((SC_CHUNK,), jnp.int32),
            pltpu.VMEM((SC_CHUNK,), jnp.int32),
            pltpu.VMEM((SC_CHUNK, 16), jnp.float32),
            pltpu.VMEM((SC_CHUNK, D_FEAT), jnp.float32),
            pltpu.SemaphoreType.DMA,
        ],
    )
    return f(posx, posy, posz, srcs, dsts, nf)


def _seg_reduce_kernel(nfe_ref, coef_ref, lid_ref, nseg_ref, cont_ref,
                       nt_ref, acc_ref,
                       r_scr, carry_scr, nprev_scr, sem):
    blk = pl.program_id(0)
    nseg = nseg_ref[0, 0, 0]
    cont = cont_ref[0, 0, 0]

    @pl.when(blk == 0)
    def _():
        carry_scr[...] = jnp.zeros((1, NV), jnp.float32)

    nfe = nfe_ref[...]
    coef = coef_ref[...]
    lid = lid_ref[0, 0, :]

    cols = []
    for c in range(12):
        cols.append(nfe * coef[:, c][:, None])
    v = jnp.concatenate(cols, axis=1)

    iota_seg = lax.broadcasted_iota(jnp.int32, (B, B), 1)
    onehot = (iota_seg == lid[:, None]).astype(jnp.float32)
    r = lax.dot_general(onehot, v, (((0,), (0,)), ((), ())),
                        preferred_element_type=jnp.float32)

    carry = carry_scr[...]
    use_carry = (cont != 0).astype(jnp.float32)
    row_iota = lax.broadcasted_iota(jnp.int32, (B, 1), 0)
    r = r + jnp.where(row_iota == 0, use_carry, 0.0) * carry

    last_mask = (lid == nseg - 1).astype(jnp.float32)[:, None]
    carry_out = jnp.sum(v * last_mask, axis=0, keepdims=True)
    carry_out = carry_out + jnp.where(nseg == 1, use_carry, 0.0) * carry
    carry_scr[...] = carry_out

    slot = lax.rem(blk, 2)
    r_scr[slot] = r

    def wait_one(j, _):
        pltpu.make_async_copy(
            r_scr.at[1 - slot, pl.ds(0, 1)],
            acc_ref.at[pl.ds(0, 1)], sem).wait()
        return 0

    @pl.when(blk > 0)
    def _():
        lax.fori_loop(0, nprev_scr[0], wait_one, 0)

    def issue_one(j, _):
        node = nt_ref[0, 0, j]
        pltpu.make_async_copy(
            r_scr.at[slot, pl.ds(j, 1)],
            acc_ref.at[pl.ds(node, 1)], sem).start()
        return 0

    lax.fori_loop(0, nseg, issue_one, 0)
    nprev_scr[0] = nseg

    @pl.when(blk == NB - 1)
    def _():
        def wait_mine(j, _):
            pltpu.make_async_copy(
                r_scr.at[slot, pl.ds(0, 1)],
                acc_ref.at[pl.ds(0, 1)], sem).wait()
            return 0
        lax.fori_loop(0, nseg, wait_mine, 0)


def _seg_reduce_phase(nfe, coef, lid3, nseg, cont, node_table):
    return pl.pallas_call(
        _seg_reduce_kernel,
        grid=(NB,),
        in_specs=[
            pl.BlockSpec((B, D_FEAT), lambda i: (i, 0)),
            pl.BlockSpec((B, 16), lambda i: (i, 0)),
            pl.BlockSpec((1, 1, B), lambda i: (i, 0, 0)),
            pl.BlockSpec((1, 1, 1), lambda i: (i, 0, 0),
                         memory_space=pltpu.SMEM),
            pl.BlockSpec((1, 1, 1), lambda i: (i, 0, 0),
                         memory_space=pltpu.SMEM),
            pl.BlockSpec((1, 1, B), lambda i: (i, 0, 0),
                         memory_space=pltpu.SMEM),
        ],
        out_specs=pl.BlockSpec(memory_space=pl.ANY),
        out_shape=jax.ShapeDtypeStruct((N_PAD, NV), jnp.float32),
        scratch_shapes=[
            pltpu.VMEM((2, B, NV), jnp.float32),
            pltpu.VMEM((1, NV), jnp.float32),
            pltpu.SMEM((1,), jnp.int32),
            pltpu.SemaphoreType.DMA,
        ],
    )(nfe, coef, lid3, nseg, cont, node_table)


def _rep_kernel(acc_ref, mask_ref, re_ref, im_ref):
    acc = acc_ref[...]
    m = mask_ref[...]
    x = acc[:, 0:512] * m
    y = acc[:, 512:1024] * m
    z = acc[:, 1024:1536] * m
    s = 0.7071067811865476
    re = jnp.concatenate([x * s, z, -x * s], axis=1)
    im = jnp.concatenate([-y * s, jnp.zeros_like(y), -y * s], axis=1)
    re_ref[...] = re.reshape(re_ref.shape)
    im_ref[...] = im.reshape(im_ref.shape)


def _rep_phase(acc, mask):
    bn = 256
    re, im = pl.pallas_call(
        _rep_kernel,
        grid=(N_PAD // bn,),
        in_specs=[
            pl.BlockSpec((bn, NV), lambda i: (i, 0)),
            pl.BlockSpec((bn, 1), lambda i: (i, 0)),
        ],
        out_specs=[
            pl.BlockSpec((bn, 3, 512), lambda i: (i, 0, 0)),
            pl.BlockSpec((bn, 3, 512), lambda i: (i, 0, 0)),
        ],
        out_shape=[
            jax.ShapeDtypeStruct((N_PAD, 3, 512), jnp.float32),
            jax.ShapeDtypeStruct((N_PAD, 3, 512), jnp.float32),
        ],
    )(acc, mask)
    return re, im


def kernel(pos, node_features, edge_idx):
    src = edge_idx[0].astype(jnp.int32)
    dst = edge_idx[1].astype(jnp.int32)

    order = jnp.argsort(src)
    src_s = src[order]
    dst_s = dst[order]
    pad = EP - N_EDGES
    srcs = jnp.concatenate([src_s, jnp.zeros((pad,), jnp.int32)])
    dsts = jnp.concatenate([dst_s, jnp.zeros((pad,), jnp.int32)])

    e_idx = jnp.arange(N_EDGES, dtype=jnp.int32)
    is_start = jnp.concatenate([
        jnp.ones((1,), jnp.bool_), src_s[1:] != src_s[:-1]])
    startb = is_start | ((e_idx % B) == 0)
    seg_cum = jnp.cumsum(startb.astype(jnp.int32))
    blk_of = e_idx // B
    fc = seg_cum[0::B]
    lid = seg_cum - fc[blk_of]
    nseg = (seg_cum[B - 1::B] - fc + 1).reshape(NB, 1, 1)
    cont = jnp.concatenate([
        jnp.zeros((1,), jnp.bool_), src_s[0::B][1:] == src_s[B - 1::B][:-1]
    ]).astype(jnp.int32).reshape(NB, 1, 1)
    node_table = jnp.zeros((NB * B,), jnp.int32).at[
        blk_of * B + lid].set(src_s).reshape(NB, 1, B)
    lid3 = lid.reshape(NB, 1, B)
    mask = jnp.zeros((N_PAD,), jnp.int32).at[src].set(1).astype(
        jnp.float32)[:, None]

    posx = jnp.pad(pos[:, 0], (0, N_PAD - N_NODES))
    posy = jnp.pad(pos[:, 1], (0, N_PAD - N_NODES))
    posz = jnp.pad(pos[:, 2], (0, N_PAD - N_NODES))
    coef, nfe = _sc_edge_phase(posx, posy, posz, srcs, dsts, node_features)

    acc = _seg_reduce_phase(nfe, coef, lid3, nseg, cont, node_table)

    re, im = _rep_phase(acc, mask)
    out = jnp.stack([re[:N_NODES], im[:N_NODES]], axis=-1)
    return out
